# baseline (speedup 1.0000x reference)
import functools

import jax
import jax.numpy as jnp
from jax import lax
from jax.experimental import pallas as pl
from jax.experimental.pallas import tpu as pltpu
from jax.experimental.pallas import tpu_sc as plsc

N = 10000
E = 160000
D = 256
B = 8
EPS = 1e-5

HW = 64
NSLAB = 4
NSC = 2
NTILE = 16
RPT = N // NTILE
EPT = E // NTILE
CH = 100
NCHUNK = EPT // CH
NBUF = 5
DW = 8

RBLK = 2000
NBLK = N // RBLK


def _scatter_slab(x4_hbm, acc_sh, sidx_v, didx_v, rows, sems,
                  accdeg_sh=None, ones_v=None):
    for b in range(NBUF):
        pltpu.async_copy(x4_hbm.at[sidx_v.at[b]], rows[b], sems[b])

    def outer(i, carry):
        for b in range(NBUF):
            j = i * NBUF + b
            pltpu.make_async_copy(x4_hbm.at[sidx_v.at[j]],
                                  rows[b], sems[b]).wait()
            pltpu.sync_copy(rows[b], acc_sh.at[didx_v.at[j]], add=True)
            if accdeg_sh is not None:
                pltpu.sync_copy(ones_v, accdeg_sh.at[didx_v.at[j]], add=True)

            @pl.when(j + NBUF < NCHUNK)
            def _():
                pltpu.async_copy(x4_hbm.at[sidx_v.at[j + NBUF]],
                                 rows[b], sems[b])
        return carry

    lax.fori_loop(0, NCHUNK // NBUF, outer, 0)


def _sc_body_a(x4_hbm, srcs_hbm, dst_hbm, zrows_hbm, zdeg_hbm, ones_hbm,
               out_hbm, deg_hbm,
               acc_sh, accdeg_sh, ones_v, sidx_v, didx_v,
               rows0, rows1, rows2, rows3, rows4,
               sem0, sem1, sem2, sem3, sem4):
    c = lax.axis_index("c")
    s = lax.axis_index("s")
    rows = [rows0, rows1, rows2, rows3, rows4]
    sems = [sem0, sem1, sem2, sem3, sem4]

    pltpu.sync_copy(ones_hbm, ones_v)
    pltpu.sync_copy(dst_hbm.at[pl.ds(s * NCHUNK, NCHUNK)], didx_v)
    pltpu.sync_copy(zrows_hbm, acc_sh.at[pl.ds(s * RPT, RPT)])
    pltpu.sync_copy(zdeg_hbm, accdeg_sh.at[pl.ds(s * RPT, RPT)])
    pltpu.sync_copy(
        srcs_hbm.at[pl.ds(c * (NTILE * NCHUNK) + s * NCHUNK, NCHUNK)], sidx_v)
    plsc.subcore_barrier()
    _scatter_slab(x4_hbm, acc_sh, sidx_v, didx_v, rows, sems,
                  accdeg_sh, ones_v)
    plsc.subcore_barrier()
    pltpu.sync_copy(acc_sh.at[pl.ds(s * RPT, RPT)],
                    out_hbm.at[pl.ds(c * N + s * RPT, RPT)])

    @pl.when(c == 0)
    def _():
        pltpu.sync_copy(accdeg_sh.at[pl.ds(s * RPT, RPT)],
                        deg_hbm.at[pl.ds(s * RPT, RPT)])


def _sc_body_b(x4_hbm, srcs_hbm, dst_hbm, zrows_hbm, out_hbm,
               acc_sh, sidx_v, didx_v,
               rows0, rows1, rows2, rows3, rows4,
               sem0, sem1, sem2, sem3, sem4):
    c = lax.axis_index("c")
    s = lax.axis_index("s")
    rows = [rows0, rows1, rows2, rows3, rows4]
    sems = [sem0, sem1, sem2, sem3, sem4]

    pltpu.sync_copy(dst_hbm.at[pl.ds(s * NCHUNK, NCHUNK)], didx_v)
    pltpu.sync_copy(zrows_hbm, acc_sh.at[pl.ds(s * RPT, RPT)])
    pltpu.sync_copy(
        srcs_hbm.at[pl.ds(c * (NTILE * NCHUNK) + s * NCHUNK, NCHUNK)], sidx_v)
    plsc.subcore_barrier()
    _scatter_slab(x4_hbm, acc_sh, sidx_v, didx_v, rows, sems)
    plsc.subcore_barrier()
    pltpu.sync_copy(acc_sh.at[pl.ds(s * RPT, RPT)],
                    out_hbm.at[pl.ds(c * N + s * RPT, RPT)])


@functools.cache
def _sc_segment_a():
    return pl.kernel(
        _sc_body_a,
        out_type=(jax.ShapeDtypeStruct((NSC * N, HW), jnp.float32),
                  jax.ShapeDtypeStruct((N, DW), jnp.float32)),
        mesh=plsc.VectorSubcoreMesh(core_axis_name="c", subcore_axis_name="s",
                                    num_cores=NSC, num_subcores=NTILE),
        scratch_types=[
            pltpu.VMEM_SHARED((N, HW), jnp.float32),
            pltpu.VMEM_SHARED((N, DW), jnp.float32),
            pltpu.VMEM((CH, DW), jnp.float32),
            pltpu.VMEM((NCHUNK, CH), jnp.int32),
            pltpu.VMEM((NCHUNK, CH), jnp.int32),
        ] + [pltpu.VMEM((CH, HW), jnp.float32)] * NBUF
          + [pltpu.SemaphoreType.DMA] * NBUF,
        compiler_params=pltpu.CompilerParams(use_tc_tiling_on_sc=False),
    )


@functools.cache
def _sc_segment_b():
    return pl.kernel(
        _sc_body_b,
        out_type=jax.ShapeDtypeStruct((NSC * N, HW), jnp.float32),
        mesh=plsc.VectorSubcoreMesh(core_axis_name="c", subcore_axis_name="s",
                                    num_cores=NSC, num_subcores=NTILE),
        scratch_types=[
            pltpu.VMEM_SHARED((N, HW), jnp.float32),
            pltpu.VMEM((NCHUNK, CH), jnp.int32),
            pltpu.VMEM((NCHUNK, CH), jnp.int32),
        ] + [pltpu.VMEM((CH, HW), jnp.float32)] * NBUF
          + [pltpu.SemaphoreType.DMA] * NBUF,
        compiler_params=pltpu.CompilerParams(use_tc_tiling_on_sc=False),
    )


def _bn_rows(t, g, b):
    m = jnp.mean(t, axis=0, keepdims=True)
    v = jnp.mean((t - m) * (t - m), axis=0, keepdims=True)
    return (t - m) * lax.rsqrt(v + EPS) * g + b


def _pm_body(xa_ref, wa_ref, part_ref):
    sw = jnp.dot(xa_ref[0, 0], wa_ref[0], preferred_element_type=jnp.float32)
    sw += jnp.dot(xa_ref[1, 0], wa_ref[1], preferred_element_type=jnp.float32)
    part_ref[...] = sw


def _pm_call(xa, wa):
    return pl.pallas_call(
        _pm_body,
        grid=(NBLK,),
        in_specs=[
            pl.BlockSpec((NSC, 1, RBLK, HW), lambda i: (0, i, 0, 0)),
            pl.BlockSpec((NSC, HW, D), lambda i: (0, 0, 0)),
        ],
        out_specs=pl.BlockSpec((RBLK, D), lambda i: (i, 0)),
        out_shape=jax.ShapeDtypeStruct((N, D), jnp.float32),
    )(xa, wa)


def _ft_body(part_ref, x_ref, d_ref, bid_ref, w_ref, bconv_ref, cw_ref,
             gamma_ref, beta_ref, vn_ref,
             w1_ref, b1_ref, g1_ref, be1_ref, w2_ref, b2_ref, g2_ref, be2_ref,
             hout_ref, vnout_ref, h2_sc, stat_sc, pooled_ref):
    p = pl.program_id(0)
    i = pl.program_id(1)

    @pl.when(p == 0)
    def _():
        deg = d_ref[:, 0]
        inv_d = 1.0 / jnp.maximum(deg, 1.0)
        sw = part_ref[...]
        for q in range(NSC):
            sw += jnp.dot(x_ref[q, 0], w_ref[q],
                          preferred_element_type=jnp.float32)
        z = (sw + deg[:, None] * cw_ref[...]) * inv_d[:, None] + bconv_ref[...]
        h2 = jnp.maximum(z, 0.0)
        h2_sc[pl.ds(i * RBLK, RBLK), :] = h2
        s1 = jnp.sum(h2, axis=0)[None, :]
        s2 = jnp.sum(h2 * h2, axis=0)[None, :]
        blk = jnp.concatenate([s1, s2, jnp.zeros((6, D), jnp.float32)], axis=0)

        @pl.when(i == 0)
        def _():
            stat_sc[...] = blk

        @pl.when(i > 0)
        def _():
            stat_sc[...] = stat_sc[...] + blk

    @pl.when(p == 1)
    def _():
        mean = stat_sc[0, :] * (1.0 / N)
        var = stat_sc[1, :] * (1.0 / N) - mean * mean
        inv = lax.rsqrt(var + EPS)
        h2 = h2_sc[pl.ds(i * RBLK, RBLK), :]
        hn = (h2 - mean[None, :]) * inv[None, :] * gamma_ref[...] + beta_ref[...]
        hout_ref[...] = hn

        bb = bid_ref[0, 0]
        rows = lax.broadcasted_iota(jnp.int32, (B, RBLK), 0)
        oh = (bb[None, :] == rows).astype(jnp.float32)
        part = jnp.dot(oh, hn, preferred_element_type=jnp.float32)

        @pl.when(i == 0)
        def _():
            pooled_ref[...] = part

        @pl.when(i > 0)
        def _():
            pooled_ref[...] = pooled_ref[...] + part

        @pl.when(i == NBLK - 1)
        def _():
            pooled = pooled_ref[...] + vn_ref[...]
            t = jnp.dot(pooled, w1_ref[...],
                        preferred_element_type=jnp.float32) + b1_ref[...]
            t = _bn_rows(t, g1_ref[...], be1_ref[...])
            t = jnp.maximum(t, 0.0)
            t = jnp.dot(t, w2_ref[...],
                        preferred_element_type=jnp.float32) + b2_ref[...]
            t = _bn_rows(t, g2_ref[...], be2_ref[...])
            t = jnp.maximum(t, 0.0)
            vnout_ref[...] = t


def _ft_call(part, xb, degm, bid3, wslab, bconv, cw, gamma, beta, vn0,
             w1, b1, g1, be1, w2, b2, g2, be2):
    row = pl.BlockSpec((1, D), lambda p, i: (0, 0))
    sq = pl.BlockSpec((D, D), lambda p, i: (0, 0))
    return pl.pallas_call(
        _ft_body,
        grid=(2, NBLK),
        in_specs=[
            pl.BlockSpec((RBLK, D), lambda p, i: (i * (1 - p), 0)),
            pl.BlockSpec((NSC, 1, RBLK, HW), lambda p, i: (0, i * (1 - p), 0, 0)),
            pl.BlockSpec((RBLK, DW), lambda p, i: (i * (1 - p), 0)),
            pl.BlockSpec((1, 1, RBLK), lambda p, i: (i * p, 0, 0)),
            pl.BlockSpec((NSC, HW, D), lambda p, i: (0, 0, 0)),
            row, row, row, row, row, sq, row, row, row, sq, row, row, row,
        ],
        out_specs=[
            pl.BlockSpec((RBLK, D), lambda p, i: (i * p, 0)),
            pl.BlockSpec((B, D), lambda p, i: (0, 0)),
        ],
        out_shape=[
            jax.ShapeDtypeStruct((N, D), jnp.float32),
            jax.ShapeDtypeStruct((B, D), jnp.float32),
        ],
        scratch_shapes=[
            pltpu.VMEM((N, D), jnp.float32),
            pltpu.VMEM((8, D), jnp.float32),
            pltpu.VMEM((B, D), jnp.float32),
        ],
    )(part, xb, degm, bid3, wslab, bconv, cw, gamma, beta, vn0,
      w1, b1, g1, be1, w2, b2, g2, be2)


def kernel(x, vn_table, W_conv, b_conv, gamma, beta, W1, b1, g1, be1,
           W2, b2, g2, be2, edge_index, batch_id):
    f32 = jnp.float32
    x4 = x.reshape(NSLAB * N, HW)
    src = edge_index[0]
    s4 = src * NSLAB
    srcs_a = jnp.concatenate([s4, s4 + 1]).reshape(NSC * NTILE * NCHUNK, CH)
    srcs_b = jnp.concatenate([s4 + 2, s4 + 3]).reshape(NSC * NTILE * NCHUNK, CH)
    dst2 = edge_index[1].reshape(NTILE * NCHUNK, CH)
    zrows = jnp.zeros((RPT, HW), f32)
    zdeg = jnp.zeros((RPT, DW), f32)
    ones8 = jnp.ones((CH, DW), f32)

    out_a, degm = _sc_segment_a()(x4, srcs_a, dst2, zrows, zdeg, ones8)
    out_b = _sc_segment_b()(x4, srcs_b, dst2, zrows)
    xa = out_a.reshape(NSC, NBLK, RBLK, HW)
    xb = out_b.reshape(NSC, NBLK, RBLK, HW)

    wslab = W_conv.reshape(NSLAB, HW, D)
    cw = vn_table @ W_conv

    part = _pm_call(xa, wslab[:NSC])

    bid3 = batch_id.reshape(NBLK, 1, RBLK)
    h_out, vn_new = _ft_call(
        part, xb, degm, bid3, wslab[NSC:], b_conv.reshape(1, D), cw,
        gamma.reshape(1, D), beta.reshape(1, D), vn_table,
        W1, b1.reshape(1, D), g1.reshape(1, D), be1.reshape(1, D),
        W2, b2.reshape(1, D), g2.reshape(1, D), be2.reshape(1, D))
    return (h_out, vn_new)


# speedup vs baseline: 1.0324x; 1.0324x over previous
import functools

import jax
import jax.numpy as jnp
from jax import lax
from jax.experimental import pallas as pl
from jax.experimental.pallas import tpu as pltpu
from jax.experimental.pallas import tpu_sc as plsc

N = 10000
E = 160000
D = 256
B = 8
EPS = 1e-5

HW = 64
NSLAB = 4
NSC = 2
NTILE = 16
RPT = N // NTILE
EPT = E // NTILE
CH = 100
NCHUNK = EPT // CH
NBUF = 5
DW = 8

RBLK = 2000
NBLK = N // RBLK


def _sc_body(x4_hbm, srcs_hbm, dst_hbm, zrows_hbm, zdeg_hbm, ones_hbm,
             out_hbm, deg_hbm,
             acc_sh, accdeg_sh, ones_v, sidx_v, didx_v,
             rows0, rows1, rows2, rows3, rows4,
             gsem0, gsem1, gsem2, gsem3, gsem4,
             ssem0, ssem1, ssem2, ssem3, ssem4,
             dsem0, dsem1, dsem2, dsem3, dsem4):
    c = lax.axis_index("c")
    s = lax.axis_index("s")
    rows = [rows0, rows1, rows2, rows3, rows4]
    gsems = [gsem0, gsem1, gsem2, gsem3, gsem4]
    ssems = [ssem0, ssem1, ssem2, ssem3, ssem4]
    dsems = [dsem0, dsem1, dsem2, dsem3, dsem4]

    pltpu.sync_copy(ones_hbm, ones_v)
    pltpu.sync_copy(dst_hbm.at[pl.ds(s * NCHUNK, NCHUNK)], didx_v)

    for p in range(NSLAB // NSC):
        slab = c + NSC * p
        pltpu.sync_copy(zrows_hbm, acc_sh.at[pl.ds(s * RPT, RPT)])
        if p == 0:
            pltpu.sync_copy(zdeg_hbm, accdeg_sh.at[pl.ds(s * RPT, RPT)])
        pltpu.sync_copy(
            srcs_hbm.at[pl.ds(slab * (NTILE * NCHUNK) + s * NCHUNK, NCHUNK)],
            sidx_v)
        plsc.subcore_barrier()

        for b in range(NBUF - 1):
            pltpu.async_copy(x4_hbm.at[sidx_v.at[b]], rows[b], gsems[b])

        def outer(i, carry):
            for b in range(NBUF):
                j = i * NBUF + b
                pltpu.make_async_copy(x4_hbm.at[sidx_v.at[j]],
                                      rows[b], gsems[b]).wait()
                pltpu.async_copy(rows[b], acc_sh.at[didx_v.at[j]], ssems[b],
                                 add=True)
                if p == 0:
                    pltpu.async_copy(ones_v, accdeg_sh.at[didx_v.at[j]],
                                     dsems[b], add=True)

                g = j + NBUF - 1
                bp = (b + NBUF - 1) % NBUF
                cond = (g < NCHUNK) if b != 0 else ((g < NCHUNK) & (j >= 1))

                @pl.when(cond)
                def _():
                    pltpu.make_async_copy(rows[bp],
                                          acc_sh.at[didx_v.at[j - 1]],
                                          ssems[bp]).wait()
                    if p == 0:
                        pltpu.make_async_copy(ones_v,
                                              accdeg_sh.at[didx_v.at[j - 1]],
                                              dsems[bp]).wait()
                    pltpu.async_copy(x4_hbm.at[sidx_v.at[g]],
                                     rows[bp], gsems[bp])

                if b == 0:
                    @pl.when(j == 0)
                    def _():
                        pltpu.async_copy(x4_hbm.at[sidx_v.at[g]],
                                         rows[bp], gsems[bp])
            return carry

        lax.fori_loop(0, NCHUNK // NBUF, outer, 0)

        for b in range(NBUF):
            jl = NCHUNK - NBUF + b
            pltpu.make_async_copy(rows[b], acc_sh.at[didx_v.at[jl]],
                                  ssems[b]).wait()
            if p == 0:
                pltpu.make_async_copy(ones_v, accdeg_sh.at[didx_v.at[jl]],
                                      dsems[b]).wait()

        plsc.subcore_barrier()
        pltpu.sync_copy(acc_sh.at[pl.ds(s * RPT, RPT)],
                        out_hbm.at[pl.ds(slab * N + s * RPT, RPT)])
        if p == 0:
            @pl.when(c == 0)
            def _():
                pltpu.sync_copy(accdeg_sh.at[pl.ds(s * RPT, RPT)],
                                deg_hbm.at[pl.ds(s * RPT, RPT)])


@functools.cache
def _sc_segment():
    return pl.kernel(
        _sc_body,
        out_type=(jax.ShapeDtypeStruct((NSLAB * N, HW), jnp.float32),
                  jax.ShapeDtypeStruct((N, DW), jnp.float32)),
        mesh=plsc.VectorSubcoreMesh(core_axis_name="c", subcore_axis_name="s",
                                    num_cores=NSC, num_subcores=NTILE),
        scratch_types=[
            pltpu.VMEM_SHARED((N, HW), jnp.float32),
            pltpu.VMEM_SHARED((N, DW), jnp.float32),
            pltpu.VMEM((CH, DW), jnp.float32),
            pltpu.VMEM((NCHUNK, CH), jnp.int32),
            pltpu.VMEM((NCHUNK, CH), jnp.int32),
        ] + [pltpu.VMEM((CH, HW), jnp.float32)] * NBUF
          + [pltpu.SemaphoreType.DMA] * (3 * NBUF),
        compiler_params=pltpu.CompilerParams(use_tc_tiling_on_sc=False),
    )


def _bn_rows(t, g, b):
    m = jnp.mean(t, axis=0, keepdims=True)
    v = jnp.mean((t - m) * (t - m), axis=0, keepdims=True)
    return (t - m) * lax.rsqrt(v + EPS) * g + b


def _ft_body(x_ref, d_ref, bid_ref, w_ref, bconv_ref, cw_ref,
             gamma_ref, beta_ref, vn_ref,
             w1_ref, b1_ref, g1_ref, be1_ref, w2_ref, b2_ref, g2_ref, be2_ref,
             hout_ref, vnout_ref, h2_sc, stat_sc, pooled_ref):
    p = pl.program_id(0)
    i = pl.program_id(1)

    @pl.when(p == 0)
    def _():
        deg = d_ref[:, 0]
        inv_d = 1.0 / jnp.maximum(deg, 1.0)
        sw = jnp.dot(x_ref[0, 0], w_ref[0], preferred_element_type=jnp.float32)
        for q in range(1, NSLAB):
            sw += jnp.dot(x_ref[q, 0], w_ref[q],
                          preferred_element_type=jnp.float32)
        z = (sw + deg[:, None] * cw_ref[...]) * inv_d[:, None] + bconv_ref[...]
        h2 = jnp.maximum(z, 0.0)
        h2_sc[pl.ds(i * RBLK, RBLK), :] = h2
        s1 = jnp.sum(h2, axis=0)[None, :]
        s2 = jnp.sum(h2 * h2, axis=0)[None, :]
        blk = jnp.concatenate([s1, s2, jnp.zeros((6, D), jnp.float32)], axis=0)

        @pl.when(i == 0)
        def _():
            stat_sc[...] = blk

        @pl.when(i > 0)
        def _():
            stat_sc[...] = stat_sc[...] + blk

    @pl.when(p == 1)
    def _():
        mean = stat_sc[0, :] * (1.0 / N)
        var = stat_sc[1, :] * (1.0 / N) - mean * mean
        inv = lax.rsqrt(var + EPS)
        h2 = h2_sc[pl.ds(i * RBLK, RBLK), :]
        hn = (h2 - mean[None, :]) * inv[None, :] * gamma_ref[...] + beta_ref[...]
        hout_ref[...] = hn

        bb = bid_ref[0, 0]
        rows = lax.broadcasted_iota(jnp.int32, (B, RBLK), 0)
        oh = (bb[None, :] == rows).astype(jnp.float32)
        part = jnp.dot(oh, hn, preferred_element_type=jnp.float32)

        @pl.when(i == 0)
        def _():
            pooled_ref[...] = part

        @pl.when(i > 0)
        def _():
            pooled_ref[...] = pooled_ref[...] + part

        @pl.when(i == NBLK - 1)
        def _():
            pooled = pooled_ref[...] + vn_ref[...]
            t = jnp.dot(pooled, w1_ref[...],
                        preferred_element_type=jnp.float32) + b1_ref[...]
            t = _bn_rows(t, g1_ref[...], be1_ref[...])
            t = jnp.maximum(t, 0.0)
            t = jnp.dot(t, w2_ref[...],
                        preferred_element_type=jnp.float32) + b2_ref[...]
            t = _bn_rows(t, g2_ref[...], be2_ref[...])
            t = jnp.maximum(t, 0.0)
            vnout_ref[...] = t


def _ft_call(xr, degm, bid3, wslab, bconv, cw, gamma, beta, vn0,
             w1, b1, g1, be1, w2, b2, g2, be2):
    row = pl.BlockSpec((1, D), lambda p, i: (0, 0))
    sq = pl.BlockSpec((D, D), lambda p, i: (0, 0))
    return pl.pallas_call(
        _ft_body,
        grid=(2, NBLK),
        in_specs=[
            pl.BlockSpec((NSLAB, 1, RBLK, HW), lambda p, i: (0, i * (1 - p), 0, 0)),
            pl.BlockSpec((RBLK, DW), lambda p, i: (i * (1 - p), 0)),
            pl.BlockSpec((1, 1, RBLK), lambda p, i: (i * p, 0, 0)),
            pl.BlockSpec((NSLAB, HW, D), lambda p, i: (0, 0, 0)),
            row, row, row, row, row, sq, row, row, row, sq, row, row, row,
        ],
        out_specs=[
            pl.BlockSpec((RBLK, D), lambda p, i: (i * p, 0)),
            pl.BlockSpec((B, D), lambda p, i: (0, 0)),
        ],
        out_shape=[
            jax.ShapeDtypeStruct((N, D), jnp.float32),
            jax.ShapeDtypeStruct((B, D), jnp.float32),
        ],
        scratch_shapes=[
            pltpu.VMEM((N, D), jnp.float32),
            pltpu.VMEM((8, D), jnp.float32),
            pltpu.VMEM((B, D), jnp.float32),
        ],
    )(xr, degm, bid3, wslab, bconv, cw, gamma, beta, vn0,
      w1, b1, g1, be1, w2, b2, g2, be2)


def kernel(x, vn_table, W_conv, b_conv, gamma, beta, W1, b1, g1, be1,
           W2, b2, g2, be2, edge_index, batch_id):
    f32 = jnp.float32
    x4 = x.reshape(NSLAB * N, HW)
    src = edge_index[0]
    s4 = src * NSLAB
    srcs = jnp.concatenate([s4, s4 + 1, s4 + 2, s4 + 3]).reshape(
        NSLAB * NTILE * NCHUNK, CH)
    dst2 = edge_index[1].reshape(NTILE * NCHUNK, CH)
    zrows = jnp.zeros((RPT, HW), f32)
    zdeg = jnp.zeros((RPT, DW), f32)
    ones8 = jnp.ones((CH, DW), f32)

    sc_out, degm = _sc_segment()(x4, srcs, dst2, zrows, zdeg, ones8)
    xr = sc_out.reshape(NSLAB, NBLK, RBLK, HW)

    wslab = W_conv.reshape(NSLAB, HW, D)
    cw = vn_table @ W_conv

    bid3 = batch_id.reshape(NBLK, 1, RBLK)
    h_out, vn_new = _ft_call(
        xr, degm, bid3, wslab, b_conv.reshape(1, D), cw,
        gamma.reshape(1, D), beta.reshape(1, D), vn_table,
        W1, b1.reshape(1, D), g1.reshape(1, D), be1.reshape(1, D),
        W2, b2.reshape(1, D), g2.reshape(1, D), be2.reshape(1, D))
    return (h_out, vn_new)


# speedup vs baseline: 1.0703x; 1.0367x over previous
import functools

import jax
import jax.numpy as jnp
from jax import lax
from jax.experimental import pallas as pl
from jax.experimental.pallas import tpu as pltpu
from jax.experimental.pallas import tpu_sc as plsc

N = 10000
E = 160000
D = 256
B = 8
EPS = 1e-5

HW = 64
NSLAB = 4
NSC = 2
NTILE = 16
RPT = N // NTILE
EPT = E // NTILE
CH = 125
NCHUNK = EPT // CH
NBUF = 5
ZR = 125
DW = 8

RBLK = 2000
NBLK = N // RBLK


def _sc_body(x4_hbm, srcs_hbm, dst_hbm, zrows_hbm, zdeg_hbm, ones_hbm,
             out_hbm, deg_hbm,
             acc_sh, accdeg_sh, ones_v, zt_v, sidx_v, didx_v,
             *ring):
    c = lax.axis_index("c")
    s = lax.axis_index("s")
    rows = list(ring[:NBUF])
    gsems = list(ring[NBUF:2 * NBUF])
    ssems = list(ring[2 * NBUF:3 * NBUF])
    dsems = list(ring[3 * NBUF:4 * NBUF])

    pltpu.sync_copy(ones_hbm, ones_v)
    pltpu.sync_copy(zrows_hbm, zt_v)
    pltpu.sync_copy(dst_hbm.at[pl.ds(s * NCHUNK, NCHUNK)], didx_v)

    for p in range(NSLAB // NSC):
        slab = c + NSC * p
        for z in range(RPT // ZR):
            pltpu.sync_copy(zt_v, acc_sh.at[pl.ds(s * RPT + z * ZR, ZR)])
        if p == 0:
            pltpu.sync_copy(zdeg_hbm, accdeg_sh.at[pl.ds(s * RPT, RPT)])
        pltpu.sync_copy(
            srcs_hbm.at[pl.ds(slab * (NTILE * NCHUNK) + s * NCHUNK, NCHUNK)],
            sidx_v)
        plsc.subcore_barrier()

        for b in range(NBUF - 1):
            pltpu.async_copy(x4_hbm.at[sidx_v.at[b]], rows[b], gsems[b])

        def outer(i, carry):
            for b in range(NBUF):
                j = i * NBUF + b
                pltpu.make_async_copy(x4_hbm.at[sidx_v.at[j]],
                                      rows[b], gsems[b]).wait()
                pltpu.async_copy(rows[b], acc_sh.at[didx_v.at[j]], ssems[b],
                                 add=True)
                if p == 0:
                    pltpu.async_copy(ones_v, accdeg_sh.at[didx_v.at[j]],
                                     dsems[b], add=True)

                g = j + NBUF - 1
                bp = (b + NBUF - 1) % NBUF
                cond = (g < NCHUNK) if b != 0 else ((g < NCHUNK) & (j >= 1))

                @pl.when(cond)
                def _():
                    pltpu.make_async_copy(rows[bp],
                                          acc_sh.at[didx_v.at[j - 1]],
                                          ssems[bp]).wait()
                    if p == 0:
                        pltpu.make_async_copy(ones_v,
                                              accdeg_sh.at[didx_v.at[j - 1]],
                                              dsems[bp]).wait()
                    pltpu.async_copy(x4_hbm.at[sidx_v.at[g]],
                                     rows[bp], gsems[bp])

                if b == 0:
                    @pl.when(j == 0)
                    def _():
                        pltpu.async_copy(x4_hbm.at[sidx_v.at[g]],
                                         rows[bp], gsems[bp])
            return carry

        lax.fori_loop(0, NCHUNK // NBUF, outer, 0)

        for b in range(NBUF):
            jl = NCHUNK - NBUF + b
            pltpu.make_async_copy(rows[b], acc_sh.at[didx_v.at[jl]],
                                  ssems[b]).wait()
            if p == 0:
                pltpu.make_async_copy(ones_v, accdeg_sh.at[didx_v.at[jl]],
                                      dsems[b]).wait()

        plsc.subcore_barrier()
        pltpu.sync_copy(acc_sh.at[pl.ds(s * RPT, RPT)],
                        out_hbm.at[pl.ds(slab * N + s * RPT, RPT)])
        if p == 0:
            @pl.when(c == 0)
            def _():
                pltpu.sync_copy(accdeg_sh.at[pl.ds(s * RPT, RPT)],
                                deg_hbm.at[pl.ds(s * RPT, RPT)])


@functools.cache
def _sc_segment():
    return pl.kernel(
        _sc_body,
        out_type=(jax.ShapeDtypeStruct((NSLAB * N, HW), jnp.float32),
                  jax.ShapeDtypeStruct((N, DW), jnp.float32)),
        mesh=plsc.VectorSubcoreMesh(core_axis_name="c", subcore_axis_name="s",
                                    num_cores=NSC, num_subcores=NTILE),
        scratch_types=[
            pltpu.VMEM_SHARED((N, HW), jnp.float32),
            pltpu.VMEM_SHARED((N, DW), jnp.float32),
            pltpu.VMEM((CH, DW), jnp.float32),
            pltpu.VMEM((ZR, HW), jnp.float32),
            pltpu.VMEM((NCHUNK, CH), jnp.int32),
            pltpu.VMEM((NCHUNK, CH), jnp.int32),
        ] + [pltpu.VMEM((CH, HW), jnp.float32)] * NBUF
          + [pltpu.SemaphoreType.DMA] * (3 * NBUF),
        compiler_params=pltpu.CompilerParams(use_tc_tiling_on_sc=False),
    )


def _bn_rows(t, g, b):
    m = jnp.mean(t, axis=0, keepdims=True)
    v = jnp.mean((t - m) * (t - m), axis=0, keepdims=True)
    return (t - m) * lax.rsqrt(v + EPS) * g + b


def _ft_body(x_ref, d_ref, bid_ref, w_ref, bconv_ref, cw_ref,
             gamma_ref, beta_ref, vn_ref,
             w1_ref, b1_ref, g1_ref, be1_ref, w2_ref, b2_ref, g2_ref, be2_ref,
             hout_ref, vnout_ref, h2_sc, stat_sc, pooled_ref):
    p = pl.program_id(0)
    i = pl.program_id(1)

    @pl.when(p == 0)
    def _():
        deg = d_ref[:, 0]
        inv_d = 1.0 / jnp.maximum(deg, 1.0)
        sw = jnp.dot(x_ref[0, 0], w_ref[0], preferred_element_type=jnp.float32)
        for q in range(1, NSLAB):
            sw += jnp.dot(x_ref[q, 0], w_ref[q],
                          preferred_element_type=jnp.float32)
        z = (sw + deg[:, None] * cw_ref[...]) * inv_d[:, None] + bconv_ref[...]
        h2 = jnp.maximum(z, 0.0)
        h2_sc[pl.ds(i * RBLK, RBLK), :] = h2
        s1 = jnp.sum(h2, axis=0)[None, :]
        s2 = jnp.sum(h2 * h2, axis=0)[None, :]
        blk = jnp.concatenate([s1, s2, jnp.zeros((6, D), jnp.float32)], axis=0)

        @pl.when(i == 0)
        def _():
            stat_sc[...] = blk

        @pl.when(i > 0)
        def _():
            stat_sc[...] = stat_sc[...] + blk

    @pl.when(p == 1)
    def _():
        mean = stat_sc[0, :] * (1.0 / N)
        var = stat_sc[1, :] * (1.0 / N) - mean * mean
        inv = lax.rsqrt(var + EPS)
        h2 = h2_sc[pl.ds(i * RBLK, RBLK), :]
        hn = (h2 - mean[None, :]) * inv[None, :] * gamma_ref[...] + beta_ref[...]
        hout_ref[...] = hn

        bb = bid_ref[0, 0]
        rows = lax.broadcasted_iota(jnp.int32, (B, RBLK), 0)
        oh = (bb[None, :] == rows).astype(jnp.float32)
        part = jnp.dot(oh, hn, preferred_element_type=jnp.float32)

        @pl.when(i == 0)
        def _():
            pooled_ref[...] = part

        @pl.when(i > 0)
        def _():
            pooled_ref[...] = pooled_ref[...] + part

        @pl.when(i == NBLK - 1)
        def _():
            pooled = pooled_ref[...] + vn_ref[...]
            t = jnp.dot(pooled, w1_ref[...],
                        preferred_element_type=jnp.float32) + b1_ref[...]
            t = _bn_rows(t, g1_ref[...], be1_ref[...])
            t = jnp.maximum(t, 0.0)
            t = jnp.dot(t, w2_ref[...],
                        preferred_element_type=jnp.float32) + b2_ref[...]
            t = _bn_rows(t, g2_ref[...], be2_ref[...])
            t = jnp.maximum(t, 0.0)
            vnout_ref[...] = t


def _ft_call(xr, degm, bid3, wslab, bconv, cw, gamma, beta, vn0,
             w1, b1, g1, be1, w2, b2, g2, be2):
    row = pl.BlockSpec((1, D), lambda p, i: (0, 0))
    sq = pl.BlockSpec((D, D), lambda p, i: (0, 0))
    return pl.pallas_call(
        _ft_body,
        grid=(2, NBLK),
        in_specs=[
            pl.BlockSpec((NSLAB, 1, RBLK, HW), lambda p, i: (0, i * (1 - p), 0, 0)),
            pl.BlockSpec((RBLK, DW), lambda p, i: (i * (1 - p), 0)),
            pl.BlockSpec((1, 1, RBLK), lambda p, i: (i * p, 0, 0)),
            pl.BlockSpec((NSLAB, HW, D), lambda p, i: (0, 0, 0)),
            row, row, row, row, row, sq, row, row, row, sq, row, row, row,
        ],
        out_specs=[
            pl.BlockSpec((RBLK, D), lambda p, i: (i * p, 0)),
            pl.BlockSpec((B, D), lambda p, i: (0, 0)),
        ],
        out_shape=[
            jax.ShapeDtypeStruct((N, D), jnp.float32),
            jax.ShapeDtypeStruct((B, D), jnp.float32),
        ],
        scratch_shapes=[
            pltpu.VMEM((N, D), jnp.float32),
            pltpu.VMEM((8, D), jnp.float32),
            pltpu.VMEM((B, D), jnp.float32),
        ],
    )(xr, degm, bid3, wslab, bconv, cw, gamma, beta, vn0,
      w1, b1, g1, be1, w2, b2, g2, be2)


def kernel(x, vn_table, W_conv, b_conv, gamma, beta, W1, b1, g1, be1,
           W2, b2, g2, be2, edge_index, batch_id):
    f32 = jnp.float32
    x4 = x.reshape(NSLAB * N, HW)
    src = edge_index[0]
    s4 = src * NSLAB
    srcs = jnp.concatenate([s4, s4 + 1, s4 + 2, s4 + 3]).reshape(
        NSLAB * NTILE * NCHUNK, CH)
    dst2 = edge_index[1].reshape(NTILE * NCHUNK, CH)
    zrows = jnp.zeros((ZR, HW), f32)
    zdeg = jnp.zeros((RPT, DW), f32)
    ones8 = jnp.ones((CH, DW), f32)

    sc_out, degm = _sc_segment()(x4, srcs, dst2, zrows, zdeg, ones8)
    xr = sc_out.reshape(NSLAB, NBLK, RBLK, HW)

    wslab = W_conv.reshape(NSLAB, HW, D)
    cw = vn_table @ W_conv

    bid3 = batch_id.reshape(NBLK, 1, RBLK)
    h_out, vn_new = _ft_call(
        xr, degm, bid3, wslab, b_conv.reshape(1, D), cw,
        gamma.reshape(1, D), beta.reshape(1, D), vn_table,
        W1, b1.reshape(1, D), g1.reshape(1, D), be1.reshape(1, D),
        W2, b2.reshape(1, D), g2.reshape(1, D), be2.reshape(1, D))
    return (h_out, vn_new)
